# baseline (speedup 1.0000x reference)
import jax
import jax.numpy as jnp
import numpy as np
from jax import lax
from jax.experimental import pallas as pl
from jax.experimental.pallas import tpu as pltpu
from jax.experimental.pallas import tpu_sc as plsc

_B, _N, _C, _H, _K = 1, 2048, 768, 12, 35
_D = _C // _H
_SCALE = 1.0 / np.sqrt(_D)

_PROJ_NB = 256
_ATTN_NB = 64

_SC_NC = 2
_SC_NS = 16
_NW = _SC_NC * _SC_NS
_NIDX = _N * _K
_BPW = _NIDX // _NW
_CH = 112


def _pack_bf16_pair(hi_f32, lo_f32):
    hi = lax.bitcast_convert_type(hi_f32, jnp.int32)
    lo = lax.bitcast_convert_type(lo_f32, jnp.int32)
    hi = (hi + 0x8000) & jnp.int32(-65536)
    lo = ((lo + 0x8000) >> 16) & jnp.int32(0xFFFF)
    return hi | lo


def _unpack_hi(packed):
    return lax.bitcast_convert_type(packed & jnp.int32(-65536), jnp.float32)


def _unpack_lo(packed):
    return lax.bitcast_convert_type(packed << 16, jnp.float32)


def _proj_body(xq_ref, xk_ref, xv_ref, wq_ref, bq_ref, wq1_ref, bq1_ref,
               wk_ref, bk_ref, wv_ref, bv_ref,
               q_ref, q1_ref, kv_ref):
    xq = xq_ref[...]
    q_ref[...] = jnp.dot(xq, wq_ref[...], preferred_element_type=jnp.float32) + bq_ref[...]
    q1_ref[...] = jnp.dot(xq, wq1_ref[...], preferred_element_type=jnp.float32) + bq1_ref[...]
    k = jnp.dot(xk_ref[...], wk_ref[...], preferred_element_type=jnp.float32) + bk_ref[...]
    v = jnp.dot(xv_ref[...], wv_ref[...], preferred_element_type=jnp.float32) + bv_ref[...]
    kv_ref[...] = _pack_bf16_pair(k, v)


def _sc_gather_body(kv_hbm, idx_hbm, out_hbm, idx_v, rows_v, sem):
    wid = lax.axis_index("s") * _SC_NC + lax.axis_index("c")
    base = wid * _BPW

    @pl.loop(0, _BPW, step=_CH)
    def _(off):
        b = base + off
        pltpu.sync_copy(idx_hbm.at[pl.ds(b, _CH)], idx_v)
        pltpu.async_copy(kv_hbm.at[idx_v], rows_v, sem).wait()
        pltpu.sync_copy(rows_v, out_hbm.at[pl.ds(b, _CH)])


def _attn_body(q_ref, q1_ref, kv_ref, rpe_ref,
               wp_ref, bp_ref, pool_ref, poolt_ref,
               hid_ref, probs_ref):
    nb = q_ref.shape[0]
    q = q_ref[...]
    q1 = q1_ref[...]
    wp = wp_ref[...]
    bp = bp_ref[...]
    pool = pool_ref[...]

    for k in range(_K):
        knb = _unpack_hi(kv_ref[:, k, :])
        rpe_k = rpe_ref[:, k, :].astype(jnp.bfloat16)
        p_k = jnp.dot(rpe_k, wp, preferred_element_type=jnp.float32) + bp
        e = jnp.dot(q * knb, pool, preferred_element_type=jnp.float32)
        ep = jnp.dot(q1 * p_k, pool, preferred_element_type=jnp.float32)
        probs_ref[:, k, :] = (e + ep) * _SCALE

    s = probs_ref[...]
    m = jnp.max(s, axis=1, keepdims=True)
    ex = jnp.exp(s - m)
    pr = ex / jnp.sum(ex, axis=1, keepdims=True)
    probs_ref[...] = pr

    acc = jnp.zeros((nb, _C), dtype=jnp.float32)
    poolt = poolt_ref[...]
    for k in range(_K):
        w = jnp.dot(pr[:, k, :], poolt, preferred_element_type=jnp.float32)
        acc = acc + w * _unpack_lo(kv_ref[:, k, :])
    hid_ref[...] = acc


def kernel(input_q, input_k, input_v, rpe_knn_embeddings, knn_idx,
           Wq, bq, Wq1, bq1, Wk, bk, Wv, bv, Wp, bp):
    xq = input_q.reshape(_N, _C)
    xk = input_k.reshape(_N, _C)
    xv = input_v.reshape(_N, _C)
    rpe = rpe_knn_embeddings.reshape(_N, _K, _C)
    idx_flat = knn_idx.reshape(_NIDX).astype(jnp.int32)

    b2 = lambda b: b.reshape(1, _C)

    q, q1, kv_packed = pl.pallas_call(
        _proj_body,
        grid=(_N // _PROJ_NB,),
        in_specs=[
            pl.BlockSpec((_PROJ_NB, _C), lambda i: (i, 0)),
            pl.BlockSpec((_PROJ_NB, _C), lambda i: (i, 0)),
            pl.BlockSpec((_PROJ_NB, _C), lambda i: (i, 0)),
        ] + [
            spec for _ in range(4) for spec in (
                pl.BlockSpec((_C, _C), lambda i: (0, 0)),
                pl.BlockSpec((1, _C), lambda i: (0, 0)),
            )
        ],
        out_specs=[pl.BlockSpec((_PROJ_NB, _C), lambda i: (i, 0))] * 3,
        out_shape=[jax.ShapeDtypeStruct((_N, _C), jnp.float32)] * 2
        + [jax.ShapeDtypeStruct((_N, _C), jnp.int32)],
    )(xq, xk, xv, Wq.T, b2(bq), Wq1.T, b2(bq1), Wk.T, b2(bk), Wv.T, b2(bv))

    mesh = plsc.VectorSubcoreMesh(core_axis_name="c", subcore_axis_name="s")
    sc_gather = pl.kernel(
        _sc_gather_body,
        mesh=mesh,
        out_type=jax.ShapeDtypeStruct((_NIDX, _C), jnp.int32),
        scratch_types=[
            pltpu.VMEM((_CH,), jnp.int32),
            pltpu.VMEM((_CH, _C), jnp.int32),
            pltpu.SemaphoreType.DMA,
        ],
    )
    kvnb = sc_gather(kv_packed, idx_flat).reshape(_N, _K, _C)

    pool = jnp.repeat(jnp.eye(_H, dtype=jnp.float32), _D, axis=0)

    hid, probs_raw = pl.pallas_call(
        _attn_body,
        grid=(_N // _ATTN_NB,),
        in_specs=[
            pl.BlockSpec((_ATTN_NB, _C), lambda i: (i, 0)),
            pl.BlockSpec((_ATTN_NB, _C), lambda i: (i, 0)),
            pl.BlockSpec((_ATTN_NB, _K, _C), lambda i: (i, 0, 0)),
            pl.BlockSpec((_ATTN_NB, _K, _C), lambda i: (i, 0, 0)),
            pl.BlockSpec((_C, _C), lambda i: (0, 0)),
            pl.BlockSpec((1, _C), lambda i: (0, 0)),
            pl.BlockSpec((_C, _H), lambda i: (0, 0)),
            pl.BlockSpec((_H, _C), lambda i: (0, 0)),
        ],
        out_specs=[
            pl.BlockSpec((_ATTN_NB, _C), lambda i: (i, 0)),
            pl.BlockSpec((_ATTN_NB, _K, _H), lambda i: (i, 0, 0)),
        ],
        out_shape=[
            jax.ShapeDtypeStruct((_N, _C), jnp.float32),
            jax.ShapeDtypeStruct((_N, _K, _H), jnp.float32),
        ],
    )(q, q1, kvnb, rpe, Wp.T.astype(jnp.bfloat16), b2(bp), pool, pool.T)

    hidden = hid.reshape(_B, _N, _C)
    attention_probs = probs_raw.transpose(0, 2, 1).reshape(_B, _N, _H, _K)
    return (hidden, attention_probs)


# speedup vs baseline: 1.0997x; 1.0997x over previous
import jax
import jax.numpy as jnp
import numpy as np
from jax import lax
from jax.experimental import pallas as pl
from jax.experimental.pallas import tpu as pltpu
from jax.experimental.pallas import tpu_sc as plsc

_B, _N, _C, _H, _K = 1, 2048, 768, 12, 35
_D = _C // _H
_KP = 40
_G = _KP // 8
_SCALE = 1.0 / np.sqrt(_D)

_PROJ_NB = 256
_ATTN_NB = 64

_SC_NC = 2
_SC_NS = 16
_NW = _SC_NC * _SC_NS
_NIDX = _N * _KP
_BPW = _NIDX // _NW
_CH = 128


def _pack_bf16_pair(hi_f32, lo_f32):
    hi = lax.bitcast_convert_type(hi_f32, jnp.int32)
    lo = lax.bitcast_convert_type(lo_f32, jnp.int32)
    hi = (hi + 0x8000) & jnp.int32(-65536)
    lo = ((lo + 0x8000) >> 16) & jnp.int32(0xFFFF)
    return hi | lo


def _unpack_hi(packed):
    return lax.bitcast_convert_type(packed & jnp.int32(-65536), jnp.float32)


def _unpack_lo(packed):
    return lax.bitcast_convert_type(packed << 16, jnp.float32)


def _proj_body(xq_ref, xk_ref, xv_ref, wq_ref, bq_ref, wq1_ref, bq1_ref,
               wk_ref, bk_ref, wv_ref, bv_ref,
               q_ref, q1_ref, kv_ref):
    xq = xq_ref[...]
    q_ref[...] = jnp.dot(xq, wq_ref[...], preferred_element_type=jnp.float32) + bq_ref[...]
    q1_ref[...] = jnp.dot(xq, wq1_ref[...], preferred_element_type=jnp.float32) + bq1_ref[...]
    k = jnp.dot(xk_ref[...], wk_ref[...], preferred_element_type=jnp.float32) + bk_ref[...]
    v = jnp.dot(xv_ref[...], wv_ref[...], preferred_element_type=jnp.float32) + bv_ref[...]
    kv_ref[...] = _pack_bf16_pair(k, v)


def _sc_gather_body(kv_hbm, idx_hbm, out_hbm, idx_v, rows_v, sem):
    wid = lax.axis_index("s") * _SC_NC + lax.axis_index("c")
    base = wid * _BPW

    @pl.loop(0, _BPW, step=_CH)
    def _(off):
        b = base + off
        pltpu.sync_copy(idx_hbm.at[pl.ds(b, _CH)], idx_v)
        pltpu.async_copy(kv_hbm.at[idx_v], rows_v, sem).wait()
        pltpu.sync_copy(rows_v, out_hbm.at[pl.ds(b, _CH)])


def _attn_body(q_ref, q1_ref, kv_ref, rpe_ref,
               wp_ref, bp_ref, pool_ref, poolt_ref,
               hid_ref, probs_ref):
    nb = q_ref.shape[0]
    q = q_ref[...]
    q1 = q1_ref[...]
    wp = wp_ref[...]
    bp = bp_ref[...]
    pool = pool_ref[...]

    def rep8(x):
        return jnp.broadcast_to(x[:, None, :], (nb, 8, _C)).reshape(nb * 8, _C)

    q_rep = rep8(q)
    q1_rep = rep8(q1)

    score_groups = []
    for g in range(_G):
        kv_g = kv_ref[:, g * 8:(g + 1) * 8, :].reshape(nb * 8, _C)
        knb = _unpack_hi(kv_g)
        if g * 8 < _K:
            w = min(8, _K - g * 8)
            rpe_g = rpe_ref[:, g * 8:g * 8 + w, :]
            if w < 8:
                rpe_g = jnp.concatenate(
                    [rpe_g, jnp.zeros((nb, 8 - w, _C), jnp.float32)], axis=1)
            rpe_g = rpe_g.reshape(nb * 8, _C).astype(jnp.bfloat16)
            p_g = jnp.dot(rpe_g, wp, preferred_element_type=jnp.float32) + bp
            ep = jnp.dot(q1_rep * p_g, pool, preferred_element_type=jnp.float32)
        else:
            ep = jnp.zeros((nb * 8, _H), jnp.float32)
        e = jnp.dot(q_rep * knb, pool, preferred_element_type=jnp.float32)
        score_groups.append(((e + ep) * _SCALE).reshape(nb, 8, _H))

    s = jnp.concatenate(score_groups, axis=1)
    kpos = jax.lax.broadcasted_iota(jnp.int32, (nb, _KP, _H), 1)
    s = jnp.where(kpos < _K, s, -1e30)
    m = jnp.max(s, axis=1, keepdims=True)
    ex = jnp.exp(s - m)
    pr = ex / jnp.sum(ex, axis=1, keepdims=True)
    probs_ref[...] = pr[:, :_K, :]

    poolt = poolt_ref[...]
    acc = jnp.zeros((nb, _C), dtype=jnp.float32)
    for g in range(_G):
        kv_g = kv_ref[:, g * 8:(g + 1) * 8, :].reshape(nb * 8, _C)
        vnb = _unpack_lo(kv_g)
        pr_g = pr[:, g * 8:(g + 1) * 8, :].reshape(nb * 8, _H)
        w = jnp.dot(pr_g, poolt, preferred_element_type=jnp.float32)
        acc = acc + jnp.sum((w * vnb).reshape(nb, 8, _C), axis=1)
    hid_ref[...] = acc


def kernel(input_q, input_k, input_v, rpe_knn_embeddings, knn_idx,
           Wq, bq, Wq1, bq1, Wk, bk, Wv, bv, Wp, bp):
    xq = input_q.reshape(_N, _C)
    xk = input_k.reshape(_N, _C)
    xv = input_v.reshape(_N, _C)
    rpe = rpe_knn_embeddings.reshape(_N, _K, _C)
    idx = knn_idx.reshape(_N, _K).astype(jnp.int32)
    idx_flat = jnp.pad(idx, ((0, 0), (0, _KP - _K))).reshape(_NIDX)

    b2 = lambda b: b.reshape(1, _C)

    q, q1, kv_packed = pl.pallas_call(
        _proj_body,
        grid=(_N // _PROJ_NB,),
        in_specs=[
            pl.BlockSpec((_PROJ_NB, _C), lambda i: (i, 0)),
            pl.BlockSpec((_PROJ_NB, _C), lambda i: (i, 0)),
            pl.BlockSpec((_PROJ_NB, _C), lambda i: (i, 0)),
        ] + [
            spec for _ in range(4) for spec in (
                pl.BlockSpec((_C, _C), lambda i: (0, 0)),
                pl.BlockSpec((1, _C), lambda i: (0, 0)),
            )
        ],
        out_specs=[pl.BlockSpec((_PROJ_NB, _C), lambda i: (i, 0))] * 3,
        out_shape=[jax.ShapeDtypeStruct((_N, _C), jnp.float32)] * 2
        + [jax.ShapeDtypeStruct((_N, _C), jnp.int32)],
    )(xq, xk, xv, Wq.T, b2(bq), Wq1.T, b2(bq1), Wk.T, b2(bk), Wv.T, b2(bv))

    mesh = plsc.VectorSubcoreMesh(core_axis_name="c", subcore_axis_name="s")
    sc_gather = pl.kernel(
        _sc_gather_body,
        mesh=mesh,
        out_type=jax.ShapeDtypeStruct((_NIDX, _C), jnp.int32),
        scratch_types=[
            pltpu.VMEM((_CH,), jnp.int32),
            pltpu.VMEM((_CH, _C), jnp.int32),
            pltpu.SemaphoreType.DMA,
        ],
    )
    kvnb = sc_gather(kv_packed, idx_flat).reshape(_N, _KP, _C)

    pool = jnp.repeat(jnp.eye(_H, dtype=jnp.float32), _D, axis=0)

    hid, probs_raw = pl.pallas_call(
        _attn_body,
        grid=(_N // _ATTN_NB,),
        in_specs=[
            pl.BlockSpec((_ATTN_NB, _C), lambda i: (i, 0)),
            pl.BlockSpec((_ATTN_NB, _C), lambda i: (i, 0)),
            pl.BlockSpec((_ATTN_NB, _KP, _C), lambda i: (i, 0, 0)),
            pl.BlockSpec((_ATTN_NB, _K, _C), lambda i: (i, 0, 0)),
            pl.BlockSpec((_C, _C), lambda i: (0, 0)),
            pl.BlockSpec((1, _C), lambda i: (0, 0)),
            pl.BlockSpec((_C, _H), lambda i: (0, 0)),
            pl.BlockSpec((_H, _C), lambda i: (0, 0)),
        ],
        out_specs=[
            pl.BlockSpec((_ATTN_NB, _C), lambda i: (i, 0)),
            pl.BlockSpec((_ATTN_NB, _K, _H), lambda i: (i, 0, 0)),
        ],
        out_shape=[
            jax.ShapeDtypeStruct((_N, _C), jnp.float32),
            jax.ShapeDtypeStruct((_N, _K, _H), jnp.float32),
        ],
    )(q, q1, kvnb, rpe, Wp.T.astype(jnp.bfloat16), b2(bp), pool, pool.T)

    hidden = hid.reshape(_B, _N, _C)
    attention_probs = probs_raw.transpose(0, 2, 1).reshape(_B, _N, _H, _K)
    return (hidden, attention_probs)
